# baseline (speedup 1.0000x reference)
import functools

import jax
import jax.numpy as jnp
from jax import lax
from jax.experimental import pallas as pl
from jax.experimental.pallas import tpu as pltpu
from jax.experimental.pallas import tpu_sc as plsc

_N = 10000
_E = 320000
_D = 128

_NC = 2
_NS = 16
_NW = _NC * _NS
_CH = 64
_NCH = 160
_HCH = 40
_EPW = _NCH * _CH
_EPAD = _NW * _EPW
_ROWS_PT = 632
_ROWS_LAST = _N - _ROWS_PT * (_NS - 1)


def _mm_body(x_ref, w_ref, o_ref):
    o_ref[...] = jnp.dot(x_ref[...], w_ref[...],
                         preferred_element_type=jnp.float32)


def _bn_body(a0_ref, a1_ref, x_ref, w2_ref, b_ref, g_ref, be_ref, o_ref):
    y = a0_ref[...] + a1_ref[...] + b_ref[...]
    y = y + jnp.dot(x_ref[...], w2_ref[...],
                    preferred_element_type=jnp.float32)
    mean = jnp.mean(y, axis=0, keepdims=True)
    yc = y - mean
    var = jnp.mean(yc * yc, axis=0, keepdims=True)
    o_ref[...] = yc * lax.rsqrt(var + 1e-5) * g_ref[...] + be_ref[...]


def _sc_body(sup_hbm, src_hbm, dst_hbm, ew_hbm,
             out0_hbm, out1_hbm,
             acc, src_v, dst_v, ew_v, gbig, gsem, ssem):
    c = lax.axis_index("c")
    s = lax.axis_index("s")
    wid = s * _NC + c

    r0 = s * _ROWS_PT
    zvec = jnp.zeros((16,), jnp.float32)

    def zrow(r, cc):
        for j in range(_D // 16):
            gbig[0, r, pl.ds(j * 16, 16)] = zvec
        return cc

    lax.fori_loop(0, _CH, zrow, 0)
    zsrc = gbig.at[0]

    @pl.when(s < _NS - 1)
    def _():
        for t in range(_ROWS_PT // _CH):
            pltpu.sync_copy(zsrc, acc.at[pl.ds(r0 + t * _CH, _CH)])
        rem = _ROWS_PT % _CH
        pltpu.sync_copy(zsrc.at[pl.ds(0, rem)],
                        acc.at[pl.ds(r0 + _ROWS_PT - rem, rem)])

    @pl.when(s == _NS - 1)
    def _():
        for t in range(_ROWS_LAST // _CH):
            pltpu.sync_copy(zsrc, acc.at[pl.ds(r0 + t * _CH, _CH)])
        rem = _ROWS_LAST % _CH
        pltpu.sync_copy(zsrc.at[pl.ds(0, rem)],
                        acc.at[pl.ds(r0 + _ROWS_LAST - rem, rem)])

    plsc.subcore_barrier()

    def buf(b):
        return gbig.at[b]

    def gather_start(g, b):
        pltpu.async_copy(sup_hbm.at[src_v.at[g]], buf(b), gsem.at[b])

    def gather_wait(g, b):
        pltpu.make_async_copy(sup_hbm.at[src_v.at[g]], buf(b),
                              gsem.at[b]).wait()

    def scatter_start(g, b):
        pltpu.async_copy(buf(b), acc.at[dst_v.at[g]], ssem.at[b], add=True)

    def scatter_wait(g, b):
        pltpu.make_async_copy(buf(b), acc.at[dst_v.at[g]], ssem.at[b]).wait()

    def scale(g, b):
        rows = buf(b)

        def group(k16, cc):
            wv = ew_v[g, pl.ds(k16 * 16, 16)]
            e0 = k16 * 16
            for k in range(16):
                w = wv[k]
                e = e0 + k
                for j in range(_D // 16):
                    sl = pl.ds(j * 16, 16)
                    rows[e, sl] = rows[e, sl] * w
            return cc

        lax.fori_loop(0, _CH // 16, group, 0)

    def stage(gq, g, k):
        b3 = (k + 3) % 4
        gather_wait(g, k)

        if k == 0:
            @pl.when(gq >= 1)
            def _():
                scatter_wait(g - 1, b3)

            gather_start(g + 3, b3)
        else:
            scatter_wait(g - 1, b3)

            @pl.when(gq <= _HCH // 4 - 2)
            def _():
                gather_start(g + 3, b3)

        scale(g, k)
        scatter_start(g, k)

    def quad(gq, carry):
        g0 = gq * 4
        for k in range(4):
            stage(gq, g0 + k, k)
        return carry

    for h in range(_NCH // _HCH):
        blk = pl.ds(h * _HCH, _HCH)
        pltpu.async_copy(src_hbm.at[wid, blk], src_v, gsem.at[0])
        pltpu.async_copy(dst_hbm.at[wid, blk], dst_v, gsem.at[1])
        pltpu.async_copy(ew_hbm.at[wid, blk], ew_v, gsem.at[2])
        pltpu.make_async_copy(src_hbm.at[wid, blk], src_v, gsem.at[0]).wait()
        pltpu.make_async_copy(dst_hbm.at[wid, blk], dst_v, gsem.at[1]).wait()
        pltpu.make_async_copy(ew_hbm.at[wid, blk], ew_v, gsem.at[2]).wait()
        gather_start(0, 0)
        gather_start(1, 1)
        gather_start(2, 2)
        lax.fori_loop(0, _HCH // 4, quad, 0)
        scatter_wait(_HCH - 1, (_HCH - 1) % 4)

    plsc.subcore_barrier()

    for cc, out_hbm in ((0, out0_hbm), (1, out1_hbm)):
        @pl.when((c == cc) & (s < _NS - 1))
        def _(out_hbm=out_hbm):
            pltpu.sync_copy(acc.at[pl.ds(r0, _ROWS_PT)],
                            out_hbm.at[pl.ds(r0, _ROWS_PT)])

        @pl.when((c == cc) & (s == _NS - 1))
        def _(out_hbm=out_hbm):
            pltpu.sync_copy(acc.at[pl.ds(r0, _ROWS_LAST)],
                            out_hbm.at[pl.ds(r0, _ROWS_LAST)])


_sc_scatter = functools.partial(
    pl.kernel,
    out_type=[jax.ShapeDtypeStruct((_N, _D), jnp.float32)] * 2,
    mesh=plsc.VectorSubcoreMesh(core_axis_name="c", subcore_axis_name="s"),
    scratch_types=[
        pltpu.VMEM_SHARED((_N, _D), jnp.float32),
        pltpu.VMEM((_HCH, _CH), jnp.int32),
        pltpu.VMEM((_HCH, _CH), jnp.int32),
        pltpu.VMEM((_HCH, _CH), jnp.float32),
        pltpu.VMEM((4, _CH, _D), jnp.float32),
        pltpu.SemaphoreType.DMA((4,)),
        pltpu.SemaphoreType.DMA((4,)),
    ],
)(_sc_body)


def kernel(x, edge_weight, weight, self_weight, bias, gamma, beta, edge_index):
    dst = edge_index[0]
    src = edge_index[1]
    pad = _EPAD - _E
    zi = (jnp.arange(pad, dtype=jnp.int32) * 16) % _N
    src_p = jnp.concatenate([src, zi]).reshape(_NW, _NCH, _CH)
    dst_p = jnp.concatenate([dst, zi]).reshape(_NW, _NCH, _CH)
    ew_p = jnp.concatenate(
        [edge_weight, jnp.zeros((pad,), jnp.float32)]).reshape(_NW, _NCH, _CH)

    sup = pl.pallas_call(
        _mm_body,
        out_shape=jax.ShapeDtypeStruct((_N, _D), jnp.float32),
    )(x, weight)

    acc0, acc1 = _sc_scatter(sup, src_p, dst_p, ew_p)

    out = pl.pallas_call(
        _bn_body,
        out_shape=jax.ShapeDtypeStruct((_N, _D), jnp.float32),
    )(acc0, acc1, x, self_weight,
      bias.reshape(1, _D), gamma.reshape(1, _D), beta.reshape(1, _D))
    return out


# speedup vs baseline: 1.0541x; 1.0541x over previous
import functools

import jax
import jax.numpy as jnp
from jax import lax
from jax.experimental import pallas as pl
from jax.experimental.pallas import tpu as pltpu
from jax.experimental.pallas import tpu_sc as plsc

_N = 10000
_E = 320000
_D = 128

_NC = 2
_NS = 16
_NW = _NC * _NS
_CH = 64
_NCH = 160
_HCH = 40
_EPW = _NCH * _CH
_EPAD = _NW * _EPW
_ROWS_PT = 632
_ROWS_LAST = _N - _ROWS_PT * (_NS - 1)


def _mm_body(x_ref, w_ref, o_ref):
    o_ref[...] = jnp.dot(x_ref[...], w_ref[...],
                         preferred_element_type=jnp.float32)


def _bn_body(a0_ref, a1_ref, x_ref, w2_ref, b_ref, g_ref, be_ref, o_ref):
    y = a0_ref[...] + a1_ref[...] + b_ref[...]
    y = y + jnp.dot(x_ref[...], w2_ref[...],
                    preferred_element_type=jnp.float32)
    mean = jnp.mean(y, axis=0, keepdims=True)
    yc = y - mean
    var = jnp.mean(yc * yc, axis=0, keepdims=True)
    o_ref[...] = yc * lax.rsqrt(var + 1e-5) * g_ref[...] + be_ref[...]


def _sc_body(sup_hbm, src_hbm, dst_hbm, ew_hbm,
             out0_hbm, out1_hbm,
             acc, src_v, dst_v, ew_v, gbig, gsem, ssem):
    c = lax.axis_index("c")
    s = lax.axis_index("s")
    wid = s * _NC + c

    r0 = s * _ROWS_PT
    zvec = jnp.zeros((16,), jnp.float32)

    def zrow(r, cc):
        for j in range(_D // 16):
            gbig[0, r, pl.ds(j * 16, 16)] = zvec
        return cc

    lax.fori_loop(0, _CH, zrow, 0)
    zsrc = gbig.at[0]

    @pl.when(s < _NS - 1)
    def _():
        for t in range(_ROWS_PT // _CH):
            pltpu.sync_copy(zsrc, acc.at[pl.ds(r0 + t * _CH, _CH)])
        rem = _ROWS_PT % _CH
        pltpu.sync_copy(zsrc.at[pl.ds(0, rem)],
                        acc.at[pl.ds(r0 + _ROWS_PT - rem, rem)])

    @pl.when(s == _NS - 1)
    def _():
        for t in range(_ROWS_LAST // _CH):
            pltpu.sync_copy(zsrc, acc.at[pl.ds(r0 + t * _CH, _CH)])
        rem = _ROWS_LAST % _CH
        pltpu.sync_copy(zsrc.at[pl.ds(0, rem)],
                        acc.at[pl.ds(r0 + _ROWS_LAST - rem, rem)])

    plsc.subcore_barrier()

    def buf(b):
        return gbig.at[b]

    def gather_start(g, b):
        pltpu.async_copy(sup_hbm.at[src_v.at[g]], buf(b), gsem.at[b])

    def gather_wait(g, b):
        pltpu.make_async_copy(sup_hbm.at[src_v.at[g]], buf(b),
                              gsem.at[b]).wait()

    def scatter_start(g, b):
        pltpu.async_copy(buf(b), acc.at[dst_v.at[g]], ssem.at[b], add=True)

    def scatter_wait(g, b):
        pltpu.make_async_copy(buf(b), acc.at[dst_v.at[g]], ssem.at[b]).wait()

    def scale(g, b):
        rows = buf(b)

        def group(k16, cc):
            wv = ew_v[g, pl.ds(k16 * 16, 16)]
            e0 = k16 * 16
            for k in range(16):
                w = wv[k]
                e = e0 + k
                for j in range(_D // 16):
                    sl = pl.ds(j * 16, 16)
                    rows[e, sl] = rows[e, sl] * w
            return cc

        lax.fori_loop(0, _CH // 16, group, 0)

    def stage(gq, g, k):
        b2 = (k + 2) % 4
        gather_wait(g, k)

        if k < 2:
            @pl.when(gq >= 1)
            def _():
                scatter_wait(g - 2, b2)

            gather_start(g + 2, b2)
        else:
            scatter_wait(g - 2, b2)

            @pl.when(gq <= _HCH // 4 - 2)
            def _():
                gather_start(g + 2, b2)

        scale(g, k)
        scatter_start(g, k)

    def quad(gq, carry):
        g0 = gq * 4
        for k in range(4):
            stage(gq, g0 + k, k)
        return carry

    for h in range(_NCH // _HCH):
        blk = pl.ds(h * _HCH, _HCH)
        pltpu.async_copy(src_hbm.at[wid, blk], src_v, gsem.at[0])
        pltpu.async_copy(dst_hbm.at[wid, blk], dst_v, gsem.at[1])
        pltpu.async_copy(ew_hbm.at[wid, blk], ew_v, gsem.at[2])
        pltpu.make_async_copy(src_hbm.at[wid, blk], src_v, gsem.at[0]).wait()
        pltpu.make_async_copy(dst_hbm.at[wid, blk], dst_v, gsem.at[1]).wait()
        pltpu.make_async_copy(ew_hbm.at[wid, blk], ew_v, gsem.at[2]).wait()
        gather_start(0, 0)
        gather_start(1, 1)
        lax.fori_loop(0, _HCH // 4, quad, 0)
        scatter_wait(_HCH - 2, (_HCH - 2) % 4)
        scatter_wait(_HCH - 1, (_HCH - 1) % 4)

    plsc.subcore_barrier()

    for cc, out_hbm in ((0, out0_hbm), (1, out1_hbm)):
        @pl.when((c == cc) & (s < _NS - 1))
        def _(out_hbm=out_hbm):
            pltpu.sync_copy(acc.at[pl.ds(r0, _ROWS_PT)],
                            out_hbm.at[pl.ds(r0, _ROWS_PT)])

        @pl.when((c == cc) & (s == _NS - 1))
        def _(out_hbm=out_hbm):
            pltpu.sync_copy(acc.at[pl.ds(r0, _ROWS_LAST)],
                            out_hbm.at[pl.ds(r0, _ROWS_LAST)])


_sc_scatter = functools.partial(
    pl.kernel,
    out_type=[jax.ShapeDtypeStruct((_N, _D), jnp.float32)] * 2,
    mesh=plsc.VectorSubcoreMesh(core_axis_name="c", subcore_axis_name="s"),
    scratch_types=[
        pltpu.VMEM_SHARED((_N, _D), jnp.float32),
        pltpu.VMEM((_HCH, _CH), jnp.int32),
        pltpu.VMEM((_HCH, _CH), jnp.int32),
        pltpu.VMEM((_HCH, _CH), jnp.float32),
        pltpu.VMEM((4, _CH, _D), jnp.float32),
        pltpu.SemaphoreType.DMA((4,)),
        pltpu.SemaphoreType.DMA((4,)),
    ],
)(_sc_body)


def kernel(x, edge_weight, weight, self_weight, bias, gamma, beta, edge_index):
    dst = edge_index[0]
    src = edge_index[1]
    pad = _EPAD - _E
    zi = (jnp.arange(pad, dtype=jnp.int32) * 16) % _N
    src_p = jnp.concatenate([src, zi]).reshape(_NW, _NCH, _CH)
    dst_p = jnp.concatenate([dst, zi]).reshape(_NW, _NCH, _CH)
    ew_p = jnp.concatenate(
        [edge_weight, jnp.zeros((pad,), jnp.float32)]).reshape(_NW, _NCH, _CH)

    sup = pl.pallas_call(
        _mm_body,
        out_shape=jax.ShapeDtypeStruct((_N, _D), jnp.float32),
    )(x, weight)

    acc0, acc1 = _sc_scatter(sup, src_p, dst_p, ew_p)

    out = pl.pallas_call(
        _bn_body,
        out_shape=jax.ShapeDtypeStruct((_N, _D), jnp.float32),
    )(acc0, acc1, x, self_weight,
      bias.reshape(1, _D), gamma.reshape(1, _D), beta.reshape(1, _D))
    return out
